# baseline (speedup 1.0000x reference)
import functools

import jax
import jax.numpy as jnp
from jax import lax
from jax.experimental import pallas as pl
from jax.experimental.pallas import tpu as pltpu
from jax.experimental.pallas import tpu_sc as plsc

_N = 10000
_E = 320000
_D = 128
_C = 40

_NC = 2
_NS = 16
_NW = _NC * _NS
_EPW = _E // _NW
_K = 40
_NCH = _EPW // _K
_NP = _N
_RPS = _NP // _NS
_ZR = 25
_ZRD = 25
_DW = 16

_f32 = jnp.float32


def _fill_rows(ref, nrows, ncols, value):
    v = jnp.full((16,), value, _f32)

    def body(r, carry):
        for c0 in range(ncols // 16):
            ref[r, pl.ds(c0 * 16, 16)] = v
        return carry

    lax.fori_loop(0, nrows, body, 0)


def _make_agg(with_deg):
    _R = 6 if with_deg else 8
    _AI = _R - 1
    _AG = _R - 2
    _G = _NCH // _R
    out_type = [jax.ShapeDtypeStruct((_NC * _NP, _D), _f32)]
    scratch = [
        pltpu.VMEM((_R, 2, _K), jnp.int32),
        pltpu.VMEM((_R, _K, _D), _f32),
        pltpu.VMEM((_ZR, _D), _f32),
        pltpu.VMEM_SHARED((_NP, _D), _f32),
        pltpu.SemaphoreType.DMA((_R,)),
        pltpu.SemaphoreType.DMA((_R,)),
        pltpu.SemaphoreType.DMA((_R,)),
    ]
    if with_deg:
        out_type.append(jax.ShapeDtypeStruct((_NC * _NP, _DW), _f32))
        scratch += [
            pltpu.VMEM((_K, _DW), _f32),
            pltpu.VMEM((_ZRD, _DW), _f32),
            pltpu.VMEM_SHARED((_NP, _DW), _f32),
            pltpu.SemaphoreType.DMA((_R,)),
        ]

    def body(g_hbm, ei_hbm, out_hbm, *rest):
        if with_deg:
            (deg_hbm, idx_v, rows_v, zero_v, acc_sh, isem, gsem, ssem,
             ones_v, zdeg_v, deg_sh, dsem) = rest
        else:
            idx_v, rows_v, zero_v, acc_sh, isem, gsem, ssem = rest
        cid = lax.axis_index("c")
        sid = lax.axis_index("s")
        wid = sid * _NC + cid
        ebase = wid * _EPW

        def idx_copy(chunk, slot):
            return pltpu.make_async_copy(
                ei_hbm.at[:, pl.ds(ebase + chunk * _K, _K)],
                idx_v.at[slot], isem.at[slot])

        def gather(slot):
            return pltpu.make_async_copy(
                g_hbm.at[idx_v.at[slot, 0]], rows_v.at[slot], gsem.at[slot])

        def scat_start(slot):
            pltpu.async_copy(rows_v.at[slot], acc_sh.at[idx_v.at[slot, 1]],
                             ssem.at[slot], add=True)
            if with_deg:
                pltpu.async_copy(ones_v, deg_sh.at[idx_v.at[slot, 1]],
                                 dsem.at[slot], add=True)

        def scat_wait(slot):
            pltpu.make_async_copy(rows_v.at[slot],
                                  acc_sh.at[idx_v.at[slot, 1]],
                                  ssem.at[slot]).wait()
            if with_deg:
                pltpu.make_async_copy(ones_v, deg_sh.at[idx_v.at[slot, 1]],
                                      dsem.at[slot]).wait()

        for s in range(_AI):
            idx_copy(s, s).start()

        _fill_rows(zero_v, _ZR, _D, 0.0)

        def zcopy(t):
            return pltpu.make_async_copy(
                zero_v, acc_sh.at[pl.ds(sid * _RPS + t * _ZR, _ZR)],
                ssem.at[0])

        def zbody(t, carry):
            zcopy(t).start()
            return carry

        lax.fori_loop(0, _RPS // _ZR, zbody, 0)

        def zwait(t, carry):
            zcopy(t).wait()
            return carry

        lax.fori_loop(0, _RPS // _ZR, zwait, 0)
        if with_deg:
            _fill_rows(ones_v, _K, _DW, 1.0)
            _fill_rows(zdeg_v, _ZRD, _DW, 0.0)

            def zdcopy(t):
                return pltpu.make_async_copy(
                    zdeg_v, deg_sh.at[pl.ds(sid * _RPS + t * _ZRD, _ZRD)],
                    ssem.at[1])

            def zdbody(t, carry):
                zdcopy(t).start()
                return carry

            lax.fori_loop(0, _RPS // _ZRD, zdbody, 0)

            def zdwait(t, carry):
                zdcopy(t).wait()
                return carry

            lax.fori_loop(0, _RPS // _ZRD, zdwait, 0)

        for s in range(_AG):
            idx_copy(s, s).wait()
            gather(s).start()
        plsc.subcore_barrier()

        def group(g, carry):
            for b in range(_R):
                jj = g * _R + b
                s_i = (b + _AI) % _R
                s_g = (b + _AG) % _R

                gather(b).wait()
                scat_start(b)

                def drain():
                    scat_wait(s_i)

                if b == 0:
                    pl.when(g > 0)(drain)
                else:
                    drain()

                def prefetch():
                    idx_copy(jj + _AI, s_i).start()

                if (_G - 1) * _R + b + _AI > _NCH - 1:
                    pl.when(g < _G - 1)(prefetch)
                else:
                    prefetch()

                def next_gather():
                    idx_copy(jj + _AG, s_g).wait()
                    gather(s_g).start()

                if (_G - 1) * _R + b + _AG > _NCH - 1:
                    pl.when(g < _G - 1)(next_gather)
                else:
                    next_gather()
            return carry

        lax.fori_loop(0, _G, group, 0)
        for c in range(_G * _R, _NCH):
            s = c % _R
            gather(s).wait()
            scat_start(s)
        for c in range(_G * _R - 1, _NCH):
            scat_wait(c % _R)
        plsc.subcore_barrier()

        r0 = sid * _RPS
        pltpu.sync_copy(acc_sh.at[pl.ds(r0, _RPS)],
                        out_hbm.at[pl.ds(cid * _NP + r0, _RPS)])
        if with_deg:
            pltpu.sync_copy(deg_sh.at[pl.ds(r0, _RPS)],
                            deg_hbm.at[pl.ds(cid * _NP + r0, _RPS)])

    mesh = plsc.VectorSubcoreMesh(core_axis_name="c", subcore_axis_name="s",
                                  num_cores=_NC, num_subcores=_NS)
    return pl.kernel(
        body,
        out_type=tuple(out_type) if with_deg else out_type[0],
        mesh=mesh,
        scratch_types=scratch,
        compiler_params=pltpu.CompilerParams(use_tc_tiling_on_sc=False),
    )


_make_agg = functools.cache(_make_agg)


_BLK = 2000
_NBLK = _NP // _BLK


def _prep_body(x_ref, w_ref, g_ref):
    g_ref[...] = jnp.dot(x_ref[...], w_ref[...], preferred_element_type=_f32)


_prep = pl.pallas_call(
    _prep_body,
    grid=(_NBLK,),
    in_specs=[
        pl.BlockSpec((_BLK, _D), lambda i: (i, 0)),
        pl.BlockSpec((_D, _D), lambda i: (0, 0)),
    ],
    out_specs=pl.BlockSpec((_BLK, _D), lambda i: (i, 0)),
    out_shape=jax.ShapeDtypeStruct((_NP, _D), _f32),
)


def _mid_body(p_ref, deg_ref, b_ref, w_ref, g_ref):
    deg = deg_ref[0, :, 0:1] + deg_ref[1, :, 0:1]
    inv = 1.0 / jnp.maximum(deg, 1.0)
    h = jnp.maximum((p_ref[0] + p_ref[1]) * inv + b_ref[...], 0.0)
    g_ref[...] = jnp.dot(h, w_ref[...], preferred_element_type=_f32)


_mid = pl.pallas_call(
    _mid_body,
    grid=(_NBLK,),
    in_specs=[
        pl.BlockSpec((_NC, _BLK, _D), lambda i: (0, i, 0)),
        pl.BlockSpec((_NC, _BLK, _DW), lambda i: (0, i, 0)),
        pl.BlockSpec((1, _D), lambda i: (0, 0)),
        pl.BlockSpec((_D, _D), lambda i: (0, 0)),
    ],
    out_specs=pl.BlockSpec((_BLK, _D), lambda i: (i, 0)),
    out_shape=jax.ShapeDtypeStruct((_NP, _D), _f32),
)


def _final_body(p_ref, deg_ref, b_ref, w_ref, bp_ref, o_ref):
    deg = deg_ref[0, :, 0:1] + deg_ref[1, :, 0:1]
    inv = 1.0 / jnp.maximum(deg, 1.0)
    h = jnp.maximum((p_ref[0] + p_ref[1]) * inv + b_ref[...], 0.0)
    o_ref[...] = jnp.dot(h, w_ref[...], preferred_element_type=_f32) + bp_ref[...]


_final = pl.pallas_call(
    _final_body,
    grid=(_NBLK,),
    in_specs=[
        pl.BlockSpec((_NC, _BLK, _D), lambda i: (0, i, 0)),
        pl.BlockSpec((_NC, _BLK, _DW), lambda i: (0, i, 0)),
        pl.BlockSpec((1, _D), lambda i: (0, 0)),
        pl.BlockSpec((_D, _C), lambda i: (0, 0)),
        pl.BlockSpec((1, _C), lambda i: (0, 0)),
    ],
    out_specs=pl.BlockSpec((_BLK, _C), lambda i: (i, 0)),
    out_shape=jax.ShapeDtypeStruct((_NP, _C), _f32),
)


def kernel(x, edge_index, W0, b0, W1, b1, W2, b2, Wp, bp):
    g0 = _prep(x, W0)
    p, deg_p = _make_agg(True)(g0, edge_index)
    p = p.reshape(_NC, _NP, _D)
    deg_p = deg_p.reshape(_NC, _NP, _DW)
    _agg = _make_agg(False)

    g1 = _mid(p, deg_p, b0.reshape(1, _D), W1)
    p = _agg(g1, edge_index).reshape(_NC, _NP, _D)

    g2 = _mid(p, deg_p, b1.reshape(1, _D), W2)
    p = _agg(g2, edge_index).reshape(_NC, _NP, _D)

    return _final(p, deg_p, b2.reshape(1, _D), Wp, bp.reshape(1, _C))


# speedup vs baseline: 1.0044x; 1.0044x over previous
import functools

import jax
import jax.numpy as jnp
from jax import lax
from jax.experimental import pallas as pl
from jax.experimental.pallas import tpu as pltpu
from jax.experimental.pallas import tpu_sc as plsc

_N = 10000
_E = 320000
_D = 128
_C = 40

_NC = 2
_NS = 16
_NW = _NC * _NS
_EPW = _E // _NW
_K = 40
_NCH = _EPW // _K
_NP = _N
_RPS = _NP // _NS
_ZR = 25
_ZRD = 25
_DW = 16

_f32 = jnp.float32


def _fill_rows(ref, nrows, ncols, value):
    v = jnp.full((16,), value, _f32)

    def body(r, carry):
        for c0 in range(ncols // 16):
            ref[r, pl.ds(c0 * 16, 16)] = v
        return carry

    lax.fori_loop(0, nrows, body, 0)


def _make_agg(with_deg):
    _R = 6 if with_deg else 8
    _AI = _R - 1
    _AG = _R - 2
    _G = _NCH // _R
    out_type = [jax.ShapeDtypeStruct((_NC * _NP, _D), _f32)]
    scratch = [
        pltpu.VMEM((_R, 2, _K), jnp.int32),
        pltpu.VMEM((_R, _K, _D), _f32),
        pltpu.VMEM((_ZR, _D), _f32),
        pltpu.VMEM_SHARED((_NP, _D), _f32),
        pltpu.SemaphoreType.DMA((_R,)),
        pltpu.SemaphoreType.DMA((_R,)),
        pltpu.SemaphoreType.DMA((_R,)),
    ]
    if with_deg:
        out_type.append(jax.ShapeDtypeStruct((_NC * _NP, _DW), _f32))
        scratch += [
            pltpu.VMEM((_K, _DW), _f32),
            pltpu.VMEM((_ZRD, _DW), _f32),
            pltpu.VMEM_SHARED((_NP, _DW), _f32),
            pltpu.SemaphoreType.DMA((_R,)),
        ]

    def body(g_hbm, ei_hbm, out_hbm, *rest):
        if with_deg:
            (deg_hbm, idx_v, rows_v, zero_v, acc_sh, isem, gsem, ssem,
             ones_v, zdeg_v, deg_sh, dsem) = rest
        else:
            idx_v, rows_v, zero_v, acc_sh, isem, gsem, ssem = rest
        cid = lax.axis_index("c")
        sid = lax.axis_index("s")
        wid = sid * _NC + cid
        ebase = wid * _EPW

        def idx_copy(chunk, slot):
            return pltpu.make_async_copy(
                ei_hbm.at[:, pl.ds(ebase + chunk * _K, _K)],
                idx_v.at[slot], isem.at[slot])

        def gather(slot):
            return pltpu.make_async_copy(
                g_hbm.at[idx_v.at[slot, 0]], rows_v.at[slot], gsem.at[slot])

        def scat_start(slot):
            pltpu.async_copy(rows_v.at[slot], acc_sh.at[idx_v.at[slot, 1]],
                             ssem.at[slot], add=True)
            if with_deg:
                pltpu.async_copy(ones_v, deg_sh.at[idx_v.at[slot, 1]],
                                 dsem.at[slot], add=True)

        def scat_wait(slot):
            pltpu.make_async_copy(rows_v.at[slot],
                                  acc_sh.at[idx_v.at[slot, 1]],
                                  ssem.at[slot]).wait()
            if with_deg:
                pltpu.make_async_copy(ones_v, deg_sh.at[idx_v.at[slot, 1]],
                                      dsem.at[slot]).wait()

        for s in range(_AI):
            idx_copy(s, s).start()

        _fill_rows(zero_v, _ZR, _D, 0.0)

        def zbody(t, carry):
            pltpu.sync_copy(zero_v, acc_sh.at[pl.ds(sid * _RPS + t * _ZR, _ZR)])
            return carry

        lax.fori_loop(0, _RPS // _ZR, zbody, 0)
        if with_deg:
            _fill_rows(ones_v, _K, _DW, 1.0)
            _fill_rows(zdeg_v, _ZRD, _DW, 0.0)

            def zdbody(t, carry):
                pltpu.sync_copy(zdeg_v,
                                deg_sh.at[pl.ds(sid * _RPS + t * _ZRD, _ZRD)])
                return carry

            lax.fori_loop(0, _RPS // _ZRD, zdbody, 0)

        for s in range(_AG):
            idx_copy(s, s).wait()
            gather(s).start()
        plsc.subcore_barrier()

        def group(g, carry):
            for b in range(_R):
                jj = g * _R + b
                s_i = (b + _AI) % _R
                s_g = (b + _AG) % _R

                gather(b).wait()
                scat_start(b)

                def drain():
                    scat_wait(s_i)

                if b == 0:
                    pl.when(g > 0)(drain)
                else:
                    drain()

                def prefetch():
                    idx_copy(jj + _AI, s_i).start()

                if (_G - 1) * _R + b + _AI > _NCH - 1:
                    pl.when(g < _G - 1)(prefetch)
                else:
                    prefetch()

                def next_gather():
                    idx_copy(jj + _AG, s_g).wait()
                    gather(s_g).start()

                if (_G - 1) * _R + b + _AG > _NCH - 1:
                    pl.when(g < _G - 1)(next_gather)
                else:
                    next_gather()
            return carry

        lax.fori_loop(0, _G, group, 0)
        for c in range(_G * _R, _NCH):
            s = c % _R
            gather(s).wait()
            scat_start(s)
        for c in range(_G * _R - 1, _NCH):
            scat_wait(c % _R)
        plsc.subcore_barrier()

        r0 = sid * _RPS
        pltpu.sync_copy(acc_sh.at[pl.ds(r0, _RPS)],
                        out_hbm.at[pl.ds(cid * _NP + r0, _RPS)])
        if with_deg:
            pltpu.sync_copy(deg_sh.at[pl.ds(r0, _RPS)],
                            deg_hbm.at[pl.ds(cid * _NP + r0, _RPS)])

    mesh = plsc.VectorSubcoreMesh(core_axis_name="c", subcore_axis_name="s",
                                  num_cores=_NC, num_subcores=_NS)
    return pl.kernel(
        body,
        out_type=tuple(out_type) if with_deg else out_type[0],
        mesh=mesh,
        scratch_types=scratch,
        compiler_params=pltpu.CompilerParams(use_tc_tiling_on_sc=False),
    )


_make_agg = functools.cache(_make_agg)


_BLK = 2000
_NBLK = _NP // _BLK


def _prep_body(x_ref, w_ref, g_ref):
    g_ref[...] = jnp.dot(x_ref[...], w_ref[...], preferred_element_type=_f32)


_prep = pl.pallas_call(
    _prep_body,
    grid=(_NBLK,),
    in_specs=[
        pl.BlockSpec((_BLK, _D), lambda i: (i, 0)),
        pl.BlockSpec((_D, _D), lambda i: (0, 0)),
    ],
    out_specs=pl.BlockSpec((_BLK, _D), lambda i: (i, 0)),
    out_shape=jax.ShapeDtypeStruct((_NP, _D), _f32),
)


def _mid_body(p_ref, deg_ref, b_ref, w_ref, g_ref):
    deg = deg_ref[0, :, 0:1] + deg_ref[1, :, 0:1]
    inv = 1.0 / jnp.maximum(deg, 1.0)
    h = jnp.maximum((p_ref[0] + p_ref[1]) * inv + b_ref[...], 0.0)
    g_ref[...] = jnp.dot(h, w_ref[...], preferred_element_type=_f32)


_mid = pl.pallas_call(
    _mid_body,
    grid=(_NBLK,),
    in_specs=[
        pl.BlockSpec((_NC, _BLK, _D), lambda i: (0, i, 0)),
        pl.BlockSpec((_NC, _BLK, _DW), lambda i: (0, i, 0)),
        pl.BlockSpec((1, _D), lambda i: (0, 0)),
        pl.BlockSpec((_D, _D), lambda i: (0, 0)),
    ],
    out_specs=pl.BlockSpec((_BLK, _D), lambda i: (i, 0)),
    out_shape=jax.ShapeDtypeStruct((_NP, _D), _f32),
)


def _final_body(p_ref, deg_ref, b_ref, w_ref, bp_ref, o_ref):
    deg = deg_ref[0, :, 0:1] + deg_ref[1, :, 0:1]
    inv = 1.0 / jnp.maximum(deg, 1.0)
    h = jnp.maximum((p_ref[0] + p_ref[1]) * inv + b_ref[...], 0.0)
    o_ref[...] = jnp.dot(h, w_ref[...], preferred_element_type=_f32) + bp_ref[...]


_final = pl.pallas_call(
    _final_body,
    grid=(_NBLK,),
    in_specs=[
        pl.BlockSpec((_NC, _BLK, _D), lambda i: (0, i, 0)),
        pl.BlockSpec((_NC, _BLK, _DW), lambda i: (0, i, 0)),
        pl.BlockSpec((1, _D), lambda i: (0, 0)),
        pl.BlockSpec((_D, _C), lambda i: (0, 0)),
        pl.BlockSpec((1, _C), lambda i: (0, 0)),
    ],
    out_specs=pl.BlockSpec((_BLK, _C), lambda i: (i, 0)),
    out_shape=jax.ShapeDtypeStruct((_NP, _C), _f32),
)


def kernel(x, edge_index, W0, b0, W1, b1, W2, b2, Wp, bp):
    g0 = _prep(x, W0)
    p, deg_p = _make_agg(True)(g0, edge_index)
    p = p.reshape(_NC, _NP, _D)
    deg_p = deg_p.reshape(_NC, _NP, _DW)
    _agg = _make_agg(False)

    g1 = _mid(p, deg_p, b0.reshape(1, _D), W1)
    p = _agg(g1, edge_index).reshape(_NC, _NP, _D)

    g2 = _mid(p, deg_p, b1.reshape(1, _D), W2)
    p = _agg(g2, edge_index).reshape(_NC, _NP, _D)

    return _final(p, deg_p, b2.reshape(1, _D), Wp, bp.reshape(1, _C))


# speedup vs baseline: 1.0089x; 1.0045x over previous
import functools

import jax
import jax.numpy as jnp
from jax import lax
from jax.experimental import pallas as pl
from jax.experimental.pallas import tpu as pltpu
from jax.experimental.pallas import tpu_sc as plsc

_N = 10000
_E = 320000
_D = 128
_C = 40

_NC = 2
_NS = 16
_NW = _NC * _NS
_EPW = _E // _NW
_K = 40
_NCH = _EPW // _K
_NP = _N
_RPS = _NP // _NS
_ZR = 25
_ZRD = 25
_DW = 16

_f32 = jnp.float32


def _fill_rows(ref, nrows, ncols, value):
    v = jnp.full((16,), value, _f32)

    def body(r, carry):
        for c0 in range(ncols // 16):
            ref[r, pl.ds(c0 * 16, 16)] = v
        return carry

    lax.fori_loop(0, nrows, body, 0)


def _make_agg(with_deg):
    _R = 7 if with_deg else 9
    _AI = _R - 1
    _AG = _R - 2
    _G = _NCH // _R
    out_type = [jax.ShapeDtypeStruct((_NC * _NP, _D), _f32)]
    scratch = [
        pltpu.VMEM((_R, 2, _K), jnp.int32),
        pltpu.VMEM((_R, _K, _D), _f32),
        pltpu.VMEM((_ZR, _D), _f32),
        pltpu.VMEM_SHARED((_NP, _D), _f32),
        pltpu.SemaphoreType.DMA((_R,)),
        pltpu.SemaphoreType.DMA((_R,)),
        pltpu.SemaphoreType.DMA((_R,)),
    ]
    if with_deg:
        out_type.append(jax.ShapeDtypeStruct((_NC * _NP, _DW), _f32))
        scratch += [
            pltpu.VMEM((_K, _DW), _f32),
            pltpu.VMEM((_ZRD, _DW), _f32),
            pltpu.VMEM_SHARED((_NP, _DW), _f32),
            pltpu.SemaphoreType.DMA((_R,)),
        ]

    def body(g_hbm, ei_hbm, out_hbm, *rest):
        if with_deg:
            (deg_hbm, idx_v, rows_v, zero_v, acc_sh, isem, gsem, ssem,
             ones_v, zdeg_v, deg_sh, dsem) = rest
        else:
            idx_v, rows_v, zero_v, acc_sh, isem, gsem, ssem = rest
        cid = lax.axis_index("c")
        sid = lax.axis_index("s")
        wid = sid * _NC + cid
        ebase = wid * _EPW

        def idx_copy(chunk, slot):
            return pltpu.make_async_copy(
                ei_hbm.at[:, pl.ds(ebase + chunk * _K, _K)],
                idx_v.at[slot], isem.at[slot])

        def gather(slot):
            return pltpu.make_async_copy(
                g_hbm.at[idx_v.at[slot, 0]], rows_v.at[slot], gsem.at[slot])

        def scat_start(slot):
            pltpu.async_copy(rows_v.at[slot], acc_sh.at[idx_v.at[slot, 1]],
                             ssem.at[slot], add=True)
            if with_deg:
                pltpu.async_copy(ones_v, deg_sh.at[idx_v.at[slot, 1]],
                                 dsem.at[slot], add=True)

        def scat_wait(slot):
            pltpu.make_async_copy(rows_v.at[slot],
                                  acc_sh.at[idx_v.at[slot, 1]],
                                  ssem.at[slot]).wait()
            if with_deg:
                pltpu.make_async_copy(ones_v, deg_sh.at[idx_v.at[slot, 1]],
                                      dsem.at[slot]).wait()

        for s in range(_AI):
            idx_copy(s, s).start()

        _fill_rows(zero_v, _ZR, _D, 0.0)

        def zbody(t, carry):
            pltpu.sync_copy(zero_v, acc_sh.at[pl.ds(sid * _RPS + t * _ZR, _ZR)])
            return carry

        lax.fori_loop(0, _RPS // _ZR, zbody, 0)
        if with_deg:
            _fill_rows(ones_v, _K, _DW, 1.0)
            _fill_rows(zdeg_v, _ZRD, _DW, 0.0)

            def zdbody(t, carry):
                pltpu.sync_copy(zdeg_v,
                                deg_sh.at[pl.ds(sid * _RPS + t * _ZRD, _ZRD)])
                return carry

            lax.fori_loop(0, _RPS // _ZRD, zdbody, 0)

        for s in range(_AG):
            idx_copy(s, s).wait()
            gather(s).start()
        plsc.subcore_barrier()

        def group(g, carry):
            for b in range(_R):
                jj = g * _R + b
                s_i = (b + _AI) % _R
                s_g = (b + _AG) % _R

                gather(b).wait()
                scat_start(b)

                def drain():
                    scat_wait(s_i)

                if b == 0:
                    pl.when(g > 0)(drain)
                else:
                    drain()

                def prefetch():
                    idx_copy(jj + _AI, s_i).start()

                if (_G - 1) * _R + b + _AI > _NCH - 1:
                    pl.when(g < _G - 1)(prefetch)
                else:
                    prefetch()

                def next_gather():
                    idx_copy(jj + _AG, s_g).wait()
                    gather(s_g).start()

                if (_G - 1) * _R + b + _AG > _NCH - 1:
                    pl.when(g < _G - 1)(next_gather)
                else:
                    next_gather()
            return carry

        lax.fori_loop(0, _G, group, 0)
        for c in range(_G * _R, _NCH):
            s = c % _R
            gather(s).wait()
            scat_start(s)
        for c in range(_G * _R - 1, _NCH):
            scat_wait(c % _R)
        plsc.subcore_barrier()

        r0 = sid * _RPS
        pltpu.sync_copy(acc_sh.at[pl.ds(r0, _RPS)],
                        out_hbm.at[pl.ds(cid * _NP + r0, _RPS)])
        if with_deg:
            pltpu.sync_copy(deg_sh.at[pl.ds(r0, _RPS)],
                            deg_hbm.at[pl.ds(cid * _NP + r0, _RPS)])

    mesh = plsc.VectorSubcoreMesh(core_axis_name="c", subcore_axis_name="s",
                                  num_cores=_NC, num_subcores=_NS)
    return pl.kernel(
        body,
        out_type=tuple(out_type) if with_deg else out_type[0],
        mesh=mesh,
        scratch_types=scratch,
        compiler_params=pltpu.CompilerParams(use_tc_tiling_on_sc=False),
    )


_make_agg = functools.cache(_make_agg)


_BLK = 2000
_NBLK = _NP // _BLK


def _prep_body(x_ref, w_ref, g_ref):
    g_ref[...] = jnp.dot(x_ref[...], w_ref[...], preferred_element_type=_f32)


_prep = pl.pallas_call(
    _prep_body,
    grid=(_NBLK,),
    in_specs=[
        pl.BlockSpec((_BLK, _D), lambda i: (i, 0)),
        pl.BlockSpec((_D, _D), lambda i: (0, 0)),
    ],
    out_specs=pl.BlockSpec((_BLK, _D), lambda i: (i, 0)),
    out_shape=jax.ShapeDtypeStruct((_NP, _D), _f32),
)


def _mid_body(p_ref, deg_ref, b_ref, w_ref, g_ref):
    deg = deg_ref[0, :, 0:1] + deg_ref[1, :, 0:1]
    inv = 1.0 / jnp.maximum(deg, 1.0)
    h = jnp.maximum((p_ref[0] + p_ref[1]) * inv + b_ref[...], 0.0)
    g_ref[...] = jnp.dot(h, w_ref[...], preferred_element_type=_f32)


_mid = pl.pallas_call(
    _mid_body,
    grid=(_NBLK,),
    in_specs=[
        pl.BlockSpec((_NC, _BLK, _D), lambda i: (0, i, 0)),
        pl.BlockSpec((_NC, _BLK, _DW), lambda i: (0, i, 0)),
        pl.BlockSpec((1, _D), lambda i: (0, 0)),
        pl.BlockSpec((_D, _D), lambda i: (0, 0)),
    ],
    out_specs=pl.BlockSpec((_BLK, _D), lambda i: (i, 0)),
    out_shape=jax.ShapeDtypeStruct((_NP, _D), _f32),
)


def _final_body(p_ref, deg_ref, b_ref, w_ref, bp_ref, o_ref):
    deg = deg_ref[0, :, 0:1] + deg_ref[1, :, 0:1]
    inv = 1.0 / jnp.maximum(deg, 1.0)
    h = jnp.maximum((p_ref[0] + p_ref[1]) * inv + b_ref[...], 0.0)
    o_ref[...] = jnp.dot(h, w_ref[...], preferred_element_type=_f32) + bp_ref[...]


_final = pl.pallas_call(
    _final_body,
    grid=(_NBLK,),
    in_specs=[
        pl.BlockSpec((_NC, _BLK, _D), lambda i: (0, i, 0)),
        pl.BlockSpec((_NC, _BLK, _DW), lambda i: (0, i, 0)),
        pl.BlockSpec((1, _D), lambda i: (0, 0)),
        pl.BlockSpec((_D, _C), lambda i: (0, 0)),
        pl.BlockSpec((1, _C), lambda i: (0, 0)),
    ],
    out_specs=pl.BlockSpec((_BLK, _C), lambda i: (i, 0)),
    out_shape=jax.ShapeDtypeStruct((_NP, _C), _f32),
)


def kernel(x, edge_index, W0, b0, W1, b1, W2, b2, Wp, bp):
    g0 = _prep(x, W0)
    p, deg_p = _make_agg(True)(g0, edge_index)
    p = p.reshape(_NC, _NP, _D)
    deg_p = deg_p.reshape(_NC, _NP, _DW)
    _agg = _make_agg(False)

    g1 = _mid(p, deg_p, b0.reshape(1, _D), W1)
    p = _agg(g1, edge_index).reshape(_NC, _NP, _D)

    g2 = _mid(p, deg_p, b1.reshape(1, _D), W2)
    p = _agg(g2, edge_index).reshape(_NC, _NP, _D)

    return _final(p, deg_p, b2.reshape(1, _D), Wp, bp.reshape(1, _C))
